# initial kernel scaffold (unmeasured)
import functools

import numpy as np
import jax
import jax.numpy as jnp
from jax import lax
from jax.experimental import pallas as pl
from jax.experimental.pallas import tpu as pltpu

N_DEV = 4
B, SQ, D = 2, 512, 1024
R = B * SQ
HL = 8
DH = 128
SCALE = 0.08838834764831843


def _rope_tables():
    inv = 1.0 / (10000.0 ** (np.arange(0, DH, 2) / DH))
    pos = np.arange(SQ)[:, None] * inv[None, :]
    cos = np.repeat(np.cos(pos), 2, axis=-1)
    sin = np.repeat(np.sin(pos), 2, axis=-1)
    cosf = np.tile(cos, (B, HL)).astype(np.float32)
    sinf = np.tile(sin, (B, HL)).astype(np.float32)
    return cosf, sinf


def _body(x_ref, wq_ref, wk_ref, wv_ref, wo_ref, cos_ref, sin_ref, out_ref,
          xg, pout, rsbuf, qb, kb, vb, ctxb,
          ag_send, ag_recv, rs_send, rs_recv):
    my = lax.axis_index("i")
    right = lax.rem(my + 1, N_DEV)
    left = lax.rem(my + N_DEV - 1, N_DEV)

    bar = pltpu.get_barrier_semaphore()
    for nbr in (left, right):
        pl.semaphore_signal(bar, inc=1, device_id=(nbr,),
                            device_id_type=pl.DeviceIdType.MESH)
    pl.semaphore_wait(bar, 2)

    xg[3] = x_ref[...]

    src_slots = (3, 0, 1)
    for h in range(N_DEV - 1):
        rdma = pltpu.make_async_remote_copy(
            src_ref=xg.at[src_slots[h]],
            dst_ref=xg.at[h],
            send_sem=ag_send.at[h],
            recv_sem=ag_recv.at[h],
            device_id=(right,),
            device_id_type=pl.DeviceIdType.MESH,
        )
        rdma.start()
        rdma.wait()

    cos = cos_ref[...]
    sin = sin_ref[...]
    lane = lax.broadcasted_iota(jnp.int32, (R, D), 1)
    even = (lane % 2) == 0

    def rope(t):
        tl = jnp.concatenate([t[:, 1:], t[:, :1]], axis=1)
        tr = jnp.concatenate([t[:, -1:], t[:, :-1]], axis=1)
        rot = jnp.where(even, -tl, tr)
        return t * cos + rot * sin

    def compute_chunk(c, _):
        xc = xg[pl.ds(c, 1)].reshape(R, D)
        q = jnp.dot(xc, wq_ref[...], preferred_element_type=jnp.float32)
        qb[...] = rope(q).astype(jnp.bfloat16)
        k = jnp.dot(xc, wk_ref[...], preferred_element_type=jnp.float32)
        kb[...] = rope(k).astype(jnp.bfloat16)
        vb[...] = jnp.dot(
            xc, wv_ref[...], preferred_element_type=jnp.float32
        ).astype(jnp.bfloat16)

        def attn_b(b, _):
            rs = pl.ds(b * SQ, SQ)
            for h in range(HL):
                cs = pl.ds(h * DH, DH)
                qh = qb[rs, cs]
                kh = kb[rs, cs]
                s = lax.dot_general(
                    qh, kh, (((1,), (1,)), ((), ())),
                    preferred_element_type=jnp.float32,
                ) * SCALE
                m = jnp.max(s, axis=1, keepdims=True)
                w = jnp.exp(s - m)
                w = w / jnp.sum(w, axis=1, keepdims=True)
                ctxh = jnp.dot(w.astype(jnp.bfloat16), vb[rs, cs],
                               preferred_element_type=jnp.float32)
                ctxb[rs, cs] = ctxh.astype(jnp.bfloat16)
            return 0

        lax.fori_loop(0, B, attn_b, 0)
        p = jnp.dot(ctxb[...], wo_ref[...],
                    preferred_element_type=jnp.float32)
        pout[pl.ds(c, 1)] = p.astype(jnp.bfloat16)[None]
        return 0

    lax.fori_loop(0, N_DEV, compute_chunk, 0)

    for s in range(N_DEV - 1):
        if s > 0:
            pout[s] = (pout[s].astype(jnp.float32)
                       + rsbuf[s - 1].astype(jnp.float32)
                       ).astype(jnp.bfloat16)
        rdma = pltpu.make_async_remote_copy(
            src_ref=pout.at[s],
            dst_ref=rsbuf.at[s],
            send_sem=rs_send.at[s],
            recv_sem=rs_recv.at[s],
            device_id=(right,),
            device_id_type=pl.DeviceIdType.MESH,
        )
        rdma.start()
        rdma.wait()

    out = pout[3].astype(jnp.float32) + rsbuf[2].astype(jnp.float32)
    out_ref[...] = out.reshape(B, SQ, D)

    @functools.partial(pl.run_scoped, sem=pltpu.SemaphoreType.REGULAR)
    def _(sem):
        for nbr in (left, right):
            pl.semaphore_signal(sem, inc=1, device_id=(nbr,),
                                device_id_type=pl.DeviceIdType.MESH)
        pl.semaphore_wait(sem, 2)


def kernel(x, Wq, Wk, Wv, Wo):
    cosf, sinf = _rope_tables()
    xb = x.astype(jnp.bfloat16).reshape(R, D)
    args = (
        xb,
        Wq.astype(jnp.bfloat16),
        Wk.astype(jnp.bfloat16),
        Wv.astype(jnp.bfloat16),
        Wo.astype(jnp.bfloat16),
        jnp.asarray(cosf),
        jnp.asarray(sinf),
    )
    return pl.pallas_call(
        _body,
        out_shape=jax.ShapeDtypeStruct((B, SQ, D), jnp.float32),
        in_specs=[pl.BlockSpec(memory_space=pltpu.VMEM)] * len(args),
        out_specs=pl.BlockSpec(memory_space=pltpu.VMEM),
        scratch_shapes=[
            pltpu.VMEM((N_DEV, R, D), jnp.bfloat16),
            pltpu.VMEM((N_DEV, R, D), jnp.bfloat16),
            pltpu.VMEM((N_DEV - 1, R, D), jnp.bfloat16),
            pltpu.VMEM((R, D), jnp.bfloat16),
            pltpu.VMEM((R, D), jnp.bfloat16),
            pltpu.VMEM((R, D), jnp.bfloat16),
            pltpu.VMEM((R, D), jnp.bfloat16),
            pltpu.SemaphoreType.DMA((N_DEV - 1,)),
            pltpu.SemaphoreType.DMA((N_DEV - 1,)),
            pltpu.SemaphoreType.DMA((N_DEV - 1,)),
            pltpu.SemaphoreType.DMA((N_DEV - 1,)),
        ],
        compiler_params=pltpu.CompilerParams(collective_id=0),
    )(*args)


# baseline (device time: 244351 ns/iter reference)
import functools

import numpy as np
import jax
import jax.numpy as jnp
from jax import lax
from jax.experimental import pallas as pl
from jax.experimental.pallas import tpu as pltpu

N_DEV = 4
B, SQ, D = 2, 512, 1024
R = B * SQ
HL = 8
DH = 128
SCALE = 0.08838834764831843


def _rope_tables():
    inv = 1.0 / (10000.0 ** (np.arange(0, DH, 2) / DH))
    pos = np.arange(SQ)[:, None] * inv[None, :]
    cos = np.repeat(np.cos(pos), 2, axis=-1)
    sin = np.repeat(np.sin(pos), 2, axis=-1)
    even = (np.arange(DH) % 2 == 0).astype(np.float64)
    sin_a = -sin * even
    sin_b = sin * (1.0 - even)
    tile = lambda t: np.tile(t, (B, HL))
    return tile(cos), tile(sin_a), tile(sin_b)


def _body(x_ref, wq_ref, wk_ref, wv_ref, wo_ref, cos_ref, sa_ref, sb_ref,
          out_ref, xg, pout, rsbuf, qb, kb, vb, ctxb,
          ag_send, ag_recv, rs_send, rs_recv):
    my = lax.axis_index("i")
    right = lax.rem(my + 1, N_DEV)
    left = lax.rem(my + N_DEV - 1, N_DEV)

    bar = pltpu.get_barrier_semaphore()
    for nbr in (left, right):
        pl.semaphore_signal(bar, inc=1, device_id=(nbr,),
                            device_id_type=pl.DeviceIdType.MESH)
    pl.semaphore_wait(bar, 2)

    xg[3] = x_ref[...]

    src_slots = (3, 0, 1)
    for h in range(N_DEV - 1):
        rdma = pltpu.make_async_remote_copy(
            src_ref=xg.at[src_slots[h]],
            dst_ref=xg.at[h],
            send_sem=ag_send.at[h],
            recv_sem=ag_recv.at[h],
            device_id=(right,),
            device_id_type=pl.DeviceIdType.MESH,
        )
        rdma.start()
        rdma.wait()

    def rope(t):
        tl = jnp.concatenate([t[:, 1:], t[:, :1]], axis=1)
        tr = jnp.concatenate([t[:, -1:], t[:, :-1]], axis=1)
        return t * cos_ref[...] + tl * sa_ref[...] + tr * sb_ref[...]

    def compute_chunk(c, _):
        xc = xg[pl.ds(c, 1)].reshape(R, D)
        q = jnp.dot(xc, wq_ref[...],
                    preferred_element_type=jnp.float32).astype(jnp.bfloat16)
        qb[...] = rope(q)
        k = jnp.dot(xc, wk_ref[...],
                    preferred_element_type=jnp.float32).astype(jnp.bfloat16)
        kb[...] = rope(k)
        vb[...] = jnp.dot(
            xc, wv_ref[...],
            preferred_element_type=jnp.float32).astype(jnp.bfloat16)

        def attn_b(b, _):
            rs = pl.ds(b * SQ, SQ)
            for h in range(HL):
                cs = pl.ds(h * DH, DH)
                qh = qb[rs, cs]
                kh = kb[rs, cs]
                s = lax.dot_general(
                    qh, kh, (((1,), (1,)), ((), ())),
                    preferred_element_type=jnp.float32,
                ) * SCALE
                m = jnp.max(s, axis=1, keepdims=True)
                w = jnp.exp(s - m)
                w = w / jnp.sum(w, axis=1, keepdims=True)
                ctxb[rs, cs] = jnp.dot(
                    w.astype(jnp.bfloat16), vb[rs, cs],
                    preferred_element_type=jnp.float32).astype(jnp.bfloat16)
            return 0

        lax.fori_loop(0, B, attn_b, 0)
        pout[pl.ds(c, 1)] = jnp.dot(
            ctxb[...], wo_ref[...], preferred_element_type=jnp.float32
        ).astype(jnp.bfloat16)[None]
        return 0

    lax.fori_loop(0, N_DEV, compute_chunk, 0)

    for s in range(N_DEV - 1):
        if s > 0:
            pout[s] = pout[s] + rsbuf[s - 1]
        rdma = pltpu.make_async_remote_copy(
            src_ref=pout.at[s],
            dst_ref=rsbuf.at[s],
            send_sem=rs_send.at[s],
            recv_sem=rs_recv.at[s],
            device_id=(right,),
            device_id_type=pl.DeviceIdType.MESH,
        )
        rdma.start()
        rdma.wait()

    out_ref[...] = (pout[3] + rsbuf[2]).reshape(B, SQ, D)

    @functools.partial(pl.run_scoped, sem=pltpu.SemaphoreType.REGULAR)
    def _(sem):
        for nbr in (left, right):
            pl.semaphore_signal(sem, inc=1, device_id=(nbr,),
                                device_id_type=pl.DeviceIdType.MESH)
        pl.semaphore_wait(sem, 2)


def kernel(x, Wq, Wk, Wv, Wo):
    cosf, sin_a, sin_b = _rope_tables()
    bf = jnp.bfloat16
    args = (
        x.astype(bf).reshape(R, D),
        Wq.astype(bf),
        Wk.astype(bf),
        Wv.astype(bf),
        Wo.astype(bf),
        jnp.asarray(cosf, dtype=bf),
        jnp.asarray(sin_a, dtype=bf),
        jnp.asarray(sin_b, dtype=bf),
    )
    out = pl.pallas_call(
        _body,
        out_shape=jax.ShapeDtypeStruct((B, SQ, D), bf),
        in_specs=[pl.BlockSpec(memory_space=pltpu.VMEM)] * len(args),
        out_specs=pl.BlockSpec(memory_space=pltpu.VMEM),
        scratch_shapes=[
            pltpu.VMEM((N_DEV, R, D), jnp.bfloat16),
            pltpu.VMEM((N_DEV, R, D), jnp.bfloat16),
            pltpu.VMEM((N_DEV - 1, R, D), jnp.bfloat16),
            pltpu.VMEM((R, D), jnp.bfloat16),
            pltpu.VMEM((R, D), jnp.bfloat16),
            pltpu.VMEM((R, D), jnp.bfloat16),
            pltpu.VMEM((R, D), jnp.bfloat16),
            pltpu.SemaphoreType.DMA((N_DEV - 1,)),
            pltpu.SemaphoreType.DMA((N_DEV - 1,)),
            pltpu.SemaphoreType.DMA((N_DEV - 1,)),
            pltpu.SemaphoreType.DMA((N_DEV - 1,)),
        ],
        compiler_params=pltpu.CompilerParams(
            collective_id=0,
            vmem_limit_bytes=100 * 1024 * 1024,
        ),
    )(*args)
    return out


if __name__ == "__main__":
    t = np.random.randn(R, D).astype(np.float64)
    cos, sa, sb = _rope_tables()
    tl = np.concatenate([t[:, 1:], t[:, :1]], axis=1)
    tr = np.concatenate([t[:, -1:], t[:, :-1]], axis=1)
    mine = t * cos + tl * sa + tr * sb

    t4 = t.reshape(B, SQ, HL, DH)
    inv = 1.0 / (10000.0 ** (np.arange(0, DH, 2) / DH))
    pos = np.arange(SQ)[:, None] * inv[None, :]
    cos_r = np.repeat(np.cos(pos), 2, axis=-1)
    sin_r = np.repeat(np.sin(pos), 2, axis=-1)
    t2 = t4.reshape(B, SQ, HL, DH // 2, 2)
    t_r = np.stack([-t2[..., 1], t2[..., 0]], axis=-1).reshape(B, SQ, HL, DH)
    ref = t4 * cos_r[None, :, None, :] + t_r * sin_r[None, :, None, :]
    print("rope table max err:", np.abs(mine - ref.reshape(R, D)).max())


# device time: 168900 ns/iter; 1.4467x vs baseline; 1.4467x over previous
import functools

import numpy as np
import jax
import jax.numpy as jnp
from jax import lax
from jax.experimental import pallas as pl
from jax.experimental.pallas import tpu as pltpu

N_DEV = 4
B, SQ, D = 2, 512, 1024
R = B * SQ
HL = 8
DH = 128
SCALE = 0.08838834764831843


def _rope_tables():
    inv = 1.0 / (10000.0 ** (np.arange(0, DH, 2) / DH))
    pos = np.arange(SQ)[:, None] * inv[None, :]
    cos = np.repeat(np.cos(pos), 2, axis=-1)
    sin = np.repeat(np.sin(pos), 2, axis=-1)
    even = (np.arange(DH) % 2 == 0).astype(np.float64)
    sin_a = -sin * even
    sin_b = sin * (1.0 - even)
    tile = lambda t: np.tile(t, (B, HL))
    return tile(cos), tile(sin_a), tile(sin_b)


def _body(x_ref, wq_ref, wk_ref, wv_ref, wo_ref, cos_ref, sa_ref, sb_ref,
          out_ref, xg, pout, rsbuf, qb, kb, vb, ctxb,
          ag_send, ag_recv, rs_send, rs_recv):
    my = lax.axis_index("i")
    right = lax.rem(my + 1, N_DEV)
    left = lax.rem(my + N_DEV - 1, N_DEV)

    bar = pltpu.get_barrier_semaphore()
    for nbr in (left, right):
        pl.semaphore_signal(bar, inc=1, device_id=(nbr,),
                            device_id_type=pl.DeviceIdType.MESH)
    pl.semaphore_wait(bar, 2)

    xg[3] = x_ref[...]

    def ag_hop(h, src_slot):
        return pltpu.make_async_remote_copy(
            src_ref=xg.at[src_slot],
            dst_ref=xg.at[h],
            send_sem=ag_send.at[h],
            recv_sem=ag_recv.at[h],
            device_id=(right,),
            device_id_type=pl.DeviceIdType.MESH,
        )

    def rs_hop(s):
        return pltpu.make_async_remote_copy(
            src_ref=pout.at[s],
            dst_ref=rsbuf.at[s],
            send_sem=rs_send.at[s],
            recv_sem=rs_recv.at[s],
            device_id=(right,),
            device_id_type=pl.DeviceIdType.MESH,
        )

    def rope(t):
        tl = jnp.concatenate([t[:, 1:], t[:, :1]], axis=1)
        tr = jnp.concatenate([t[:, -1:], t[:, :-1]], axis=1)
        return t * cos_ref[...] + tl * sa_ref[...] + tr * sb_ref[...]

    def compute_chunk(c):
        xc = xg[c]
        q = jnp.dot(xc, wq_ref[...],
                    preferred_element_type=jnp.float32).astype(jnp.bfloat16)
        qb[...] = rope(q)
        k = jnp.dot(xc, wk_ref[...],
                    preferred_element_type=jnp.float32).astype(jnp.bfloat16)
        kb[...] = rope(k)
        vb[...] = jnp.dot(
            xc, wv_ref[...],
            preferred_element_type=jnp.float32).astype(jnp.bfloat16)

        def attn_b(b, _):
            rs = pl.ds(b * SQ, SQ)
            for h in range(HL):
                cs = pl.ds(h * DH, DH)
                qh = qb[rs, cs]
                kh = kb[rs, cs]
                s = lax.dot_general(
                    qh, kh, (((1,), (1,)), ((), ())),
                    preferred_element_type=jnp.float32,
                ) * SCALE
                m = jnp.max(s, axis=1, keepdims=True)
                w = jnp.exp(s - m)
                w = w / jnp.sum(w, axis=1, keepdims=True)
                ctxb[rs, cs] = jnp.dot(
                    w.astype(jnp.bfloat16), vb[rs, cs],
                    preferred_element_type=jnp.float32).astype(jnp.bfloat16)
            return 0

        lax.fori_loop(0, B, attn_b, 0)
        pout[c] = jnp.dot(
            ctxb[...], wo_ref[...], preferred_element_type=jnp.float32
        ).astype(jnp.bfloat16)

    ag0 = ag_hop(0, 3)
    ag0.start()
    compute_chunk(3)

    ag0.wait_recv()
    ag1 = ag_hop(1, 0)
    ag1.start()
    compute_chunk(0)
    rs0 = rs_hop(0)
    rs0.start()

    ag1.wait_recv()
    ag2 = ag_hop(2, 1)
    ag2.start()
    compute_chunk(1)
    rs0.wait_recv()
    pout[1] = pout[1] + rsbuf[0]
    rs1 = rs_hop(1)
    rs1.start()

    ag2.wait_recv()
    compute_chunk(2)
    rs1.wait_recv()
    pout[2] = pout[2] + rsbuf[1]
    rs2 = rs_hop(2)
    rs2.start()

    rs2.wait_recv()
    out_ref[...] = (pout[3] + rsbuf[2]).reshape(B, SQ, D)

    for d in (ag0, ag1, ag2, rs0, rs1, rs2):
        d.wait_send()

    @functools.partial(pl.run_scoped, sem=pltpu.SemaphoreType.REGULAR)
    def _(sem):
        for nbr in (left, right):
            pl.semaphore_signal(sem, inc=1, device_id=(nbr,),
                                device_id_type=pl.DeviceIdType.MESH)
        pl.semaphore_wait(sem, 2)


def kernel(x, Wq, Wk, Wv, Wo):
    cosf, sin_a, sin_b = _rope_tables()
    bf = jnp.bfloat16
    args = (
        x.astype(bf).reshape(R, D),
        Wq.astype(bf),
        Wk.astype(bf),
        Wv.astype(bf),
        Wo.astype(bf),
        jnp.asarray(cosf, dtype=bf),
        jnp.asarray(sin_a, dtype=bf),
        jnp.asarray(sin_b, dtype=bf),
    )
    out = pl.pallas_call(
        _body,
        out_shape=jax.ShapeDtypeStruct((B, SQ, D), bf),
        in_specs=[pl.BlockSpec(memory_space=pltpu.VMEM)] * len(args),
        out_specs=pl.BlockSpec(memory_space=pltpu.VMEM),
        scratch_shapes=[
            pltpu.VMEM((N_DEV, R, D), jnp.bfloat16),
            pltpu.VMEM((N_DEV, R, D), jnp.bfloat16),
            pltpu.VMEM((N_DEV - 1, R, D), jnp.bfloat16),
            pltpu.VMEM((R, D), jnp.bfloat16),
            pltpu.VMEM((R, D), jnp.bfloat16),
            pltpu.VMEM((R, D), jnp.bfloat16),
            pltpu.VMEM((R, D), jnp.bfloat16),
            pltpu.SemaphoreType.DMA((N_DEV - 1,)),
            pltpu.SemaphoreType.DMA((N_DEV - 1,)),
            pltpu.SemaphoreType.DMA((N_DEV - 1,)),
            pltpu.SemaphoreType.DMA((N_DEV - 1,)),
        ],
        compiler_params=pltpu.CompilerParams(
            collective_id=0,
            vmem_limit_bytes=100 * 1024 * 1024,
        ),
    )(*args)
    return out


if __name__ == "__main__":
    t = np.random.randn(R, D).astype(np.float64)
    cos, sa, sb = _rope_tables()
    tl = np.concatenate([t[:, 1:], t[:, :1]], axis=1)
    tr = np.concatenate([t[:, -1:], t[:, :-1]], axis=1)
    mine = t * cos + tl * sa + tr * sb

    t4 = t.reshape(B, SQ, HL, DH)
    inv = 1.0 / (10000.0 ** (np.arange(0, DH, 2) / DH))
    pos = np.arange(SQ)[:, None] * inv[None, :]
    cos_r = np.repeat(np.cos(pos), 2, axis=-1)
    sin_r = np.repeat(np.sin(pos), 2, axis=-1)
    t2 = t4.reshape(B, SQ, HL, DH // 2, 2)
    t_r = np.stack([-t2[..., 1], t2[..., 0]], axis=-1).reshape(B, SQ, HL, DH)
    ref = t4 * cos_r[None, :, None, :] + t_r * sin_r[None, :, None, :]
    print("rope table max err:", np.abs(mine - ref.reshape(R, D)).max())


# device time: 143179 ns/iter; 1.7066x vs baseline; 1.1796x over previous
import functools

import numpy as np
import jax
import jax.numpy as jnp
from jax import lax
from jax.experimental import pallas as pl
from jax.experimental.pallas import tpu as pltpu

N_DEV = 4
B, SQ, D = 2, 512, 1024
R = B * SQ
HL = 8
DH = 128
SCALE = 0.08838834764831843


def _rope_tables():
    inv = 1.0 / (10000.0 ** (np.arange(0, DH, 2) / DH))
    pos = np.arange(SQ)[:, None] * inv[None, :]
    cos = np.repeat(np.cos(pos), 2, axis=-1)
    sin = np.repeat(np.sin(pos), 2, axis=-1)
    even = (np.arange(DH) % 2 == 0).astype(np.float64)
    sin_a = -sin * even
    sin_b = sin * (1.0 - even)
    tile = lambda t: np.tile(t, (B, HL))
    return tile(cos), tile(sin_a), tile(sin_b)


def _body(x_ref, wq_ref, wk_ref, wv_ref, wo_ref, cos_ref, sa_ref, sb_ref,
          out_ref, xg, pout, rsbuf, qb, kb, vb, ctxb,
          ag_send, ag_recv, rs_send, rs_recv):
    my = lax.axis_index("i")
    right = lax.rem(my + 1, N_DEV)
    left = lax.rem(my + N_DEV - 1, N_DEV)

    bar = pltpu.get_barrier_semaphore()
    for nbr in (left, right):
        pl.semaphore_signal(bar, inc=1, device_id=(nbr,),
                            device_id_type=pl.DeviceIdType.MESH)
    pl.semaphore_wait(bar, 2)

    xg[3] = x_ref[...]

    def ag_hop(h, src_slot, dev):
        return pltpu.make_async_remote_copy(
            src_ref=xg.at[src_slot],
            dst_ref=xg.at[h],
            send_sem=ag_send.at[h],
            recv_sem=ag_recv.at[h],
            device_id=(dev,),
            device_id_type=pl.DeviceIdType.MESH,
        )

    def rs_hop(s, src_slot):
        return pltpu.make_async_remote_copy(
            src_ref=pout.at[src_slot],
            dst_ref=rsbuf.at[s],
            send_sem=rs_send.at[s],
            recv_sem=rs_recv.at[s],
            device_id=(left,),
            device_id_type=pl.DeviceIdType.MESH,
        )

    def rope(t):
        tl = jnp.concatenate([t[:, 1:], t[:, :1]], axis=1)
        tr = jnp.concatenate([t[:, -1:], t[:, :-1]], axis=1)
        return t * cos_ref[...] + tl * sa_ref[...] + tr * sb_ref[...]

    def compute_chunk(c):
        xc = xg[c]
        q = jnp.dot(xc, wq_ref[...],
                    preferred_element_type=jnp.float32).astype(jnp.bfloat16)
        qb[...] = rope(q)
        k = jnp.dot(xc, wk_ref[...],
                    preferred_element_type=jnp.float32).astype(jnp.bfloat16)
        kb[...] = rope(k)
        vb[...] = jnp.dot(
            xc, wv_ref[...],
            preferred_element_type=jnp.float32).astype(jnp.bfloat16)

        def attn_b(b, _):
            rs = pl.ds(b * SQ, SQ)
            for h in range(HL):
                cs = pl.ds(h * DH, DH)
                qh = qb[rs, cs]
                kh = kb[rs, cs]
                s = lax.dot_general(
                    qh, kh, (((1,), (1,)), ((), ())),
                    preferred_element_type=jnp.float32,
                ) * SCALE
                m = jnp.max(s, axis=1, keepdims=True)
                w = jnp.exp(s - m)
                r = 1.0 / jnp.sum(w, axis=1, keepdims=True)
                ctx = jnp.dot(w.astype(jnp.bfloat16), vb[rs, cs],
                              preferred_element_type=jnp.float32)
                ctxb[rs, cs] = (ctx * r).astype(jnp.bfloat16)
            return 0

        lax.fori_loop(0, B, attn_b, 0)
        pout[c] = jnp.dot(
            ctxb[...], wo_ref[...], preferred_element_type=jnp.float32
        ).astype(jnp.bfloat16)

    ag_a = ag_hop(0, 3, right)
    ag_b = ag_hop(1, 3, left)
    ag_a.start()
    ag_b.start()
    compute_chunk(3)

    ag_a.wait_recv()
    ag_c = ag_hop(2, 0, right)
    ag_c.start()
    ag_b.wait_recv()
    compute_chunk(1)
    rs0 = rs_hop(0, 1)
    rs0.start()

    ag_c.wait_recv()
    compute_chunk(2)
    rs0.wait_recv()
    pout[2] = pout[2] + rsbuf[0]
    rs1 = rs_hop(1, 2)
    rs1.start()

    compute_chunk(0)
    rs1.wait_recv()
    pout[0] = pout[0] + rsbuf[1]
    rs2 = rs_hop(2, 0)
    rs2.start()

    rs2.wait_recv()
    out_ref[...] = (pout[3] + rsbuf[2]).reshape(B, SQ, D)

    for d in (ag_a, ag_b, ag_c, rs0, rs1, rs2):
        d.wait_send()

    @functools.partial(pl.run_scoped, sem=pltpu.SemaphoreType.REGULAR)
    def _(sem):
        for nbr in (left, right):
            pl.semaphore_signal(sem, inc=1, device_id=(nbr,),
                                device_id_type=pl.DeviceIdType.MESH)
        pl.semaphore_wait(sem, 2)


def kernel(x, Wq, Wk, Wv, Wo):
    cosf, sin_a, sin_b = _rope_tables()
    bf = jnp.bfloat16
    args = (
        x.astype(bf).reshape(R, D),
        Wq.astype(bf),
        Wk.astype(bf),
        Wv.astype(bf),
        Wo.astype(bf),
        jnp.asarray(cosf, dtype=bf),
        jnp.asarray(sin_a, dtype=bf),
        jnp.asarray(sin_b, dtype=bf),
    )
    out = pl.pallas_call(
        _body,
        out_shape=jax.ShapeDtypeStruct((B, SQ, D), bf),
        in_specs=[pl.BlockSpec(memory_space=pltpu.VMEM)] * len(args),
        out_specs=pl.BlockSpec(memory_space=pltpu.VMEM),
        scratch_shapes=[
            pltpu.VMEM((N_DEV, R, D), jnp.bfloat16),
            pltpu.VMEM((N_DEV, R, D), jnp.bfloat16),
            pltpu.VMEM((N_DEV - 1, R, D), jnp.bfloat16),
            pltpu.VMEM((R, D), jnp.bfloat16),
            pltpu.VMEM((R, D), jnp.bfloat16),
            pltpu.VMEM((R, D), jnp.bfloat16),
            pltpu.VMEM((R, D), jnp.bfloat16),
            pltpu.SemaphoreType.DMA((N_DEV - 1,)),
            pltpu.SemaphoreType.DMA((N_DEV - 1,)),
            pltpu.SemaphoreType.DMA((N_DEV - 1,)),
            pltpu.SemaphoreType.DMA((N_DEV - 1,)),
        ],
        compiler_params=pltpu.CompilerParams(
            collective_id=0,
            vmem_limit_bytes=100 * 1024 * 1024,
        ),
    )(*args)
    return out


if __name__ == "__main__":
    t = np.random.randn(R, D).astype(np.float64)
    cos, sa, sb = _rope_tables()
    tl = np.concatenate([t[:, 1:], t[:, :1]], axis=1)
    tr = np.concatenate([t[:, -1:], t[:, :-1]], axis=1)
    mine = t * cos + tl * sa + tr * sb

    t4 = t.reshape(B, SQ, HL, DH)
    inv = 1.0 / (10000.0 ** (np.arange(0, DH, 2) / DH))
    pos = np.arange(SQ)[:, None] * inv[None, :]
    cos_r = np.repeat(np.cos(pos), 2, axis=-1)
    sin_r = np.repeat(np.sin(pos), 2, axis=-1)
    t2 = t4.reshape(B, SQ, HL, DH // 2, 2)
    t_r = np.stack([-t2[..., 1], t2[..., 0]], axis=-1).reshape(B, SQ, HL, DH)
    ref = t4 * cos_r[None, :, None, :] + t_r * sin_r[None, :, None, :]
    print("rope table max err:", np.abs(mine - ref.reshape(R, D)).max())


# device time: 140677 ns/iter; 1.7370x vs baseline; 1.0178x over previous
import functools

import numpy as np
import jax
import jax.numpy as jnp
from jax import lax
from jax.experimental import pallas as pl
from jax.experimental.pallas import tpu as pltpu

N_DEV = 4
B, SQ, D = 2, 512, 1024
R = B * SQ
HL = 8
DH = 128
SCALE = 0.08838834764831843


def _rope_tables():
    inv = 1.0 / (10000.0 ** (np.arange(0, DH, 2) / DH))
    pos = np.arange(SQ)[:, None] * inv[None, :]
    cos = np.repeat(np.cos(pos), 2, axis=-1)
    sin = np.repeat(np.sin(pos), 2, axis=-1)
    even = (np.arange(DH) % 2 == 0).astype(np.float64)
    sin_a = -sin * even
    sin_b = sin * (1.0 - even)
    tile = lambda t: np.tile(t, (B, HL))
    return tile(cos), tile(sin_a), tile(sin_b)


def _body(x_ref, wq_ref, wk_ref, wv_ref, wo_ref, cos_ref, sa_ref, sb_ref,
          out_ref, xg, pout, rsbuf, qb, kb, vb, ctxb,
          ag_send, ag_recv, rs_send, rs_recv):
    my = lax.axis_index("i")
    right = lax.rem(my + 1, N_DEV)
    left = lax.rem(my + N_DEV - 1, N_DEV)

    bar = pltpu.get_barrier_semaphore()
    for nbr in (left, right):
        pl.semaphore_signal(bar, inc=1, device_id=(nbr,),
                            device_id_type=pl.DeviceIdType.MESH)
    pl.semaphore_wait(bar, 2)

    xg[3] = x_ref[...]

    def ag_hop(h, src_slot, dev):
        return pltpu.make_async_remote_copy(
            src_ref=xg.at[src_slot],
            dst_ref=xg.at[h],
            send_sem=ag_send.at[h],
            recv_sem=ag_recv.at[h],
            device_id=(dev,),
            device_id_type=pl.DeviceIdType.MESH,
        )

    def rs_hop(s, src_slot):
        return pltpu.make_async_remote_copy(
            src_ref=pout.at[src_slot],
            dst_ref=rsbuf.at[s],
            send_sem=rs_send.at[s],
            recv_sem=rs_recv.at[s],
            device_id=(left,),
            device_id_type=pl.DeviceIdType.MESH,
        )

    def rope(t):
        tl = jnp.concatenate([t[:, 1:], t[:, :1]], axis=1)
        tr = jnp.concatenate([t[:, -1:], t[:, :-1]], axis=1)
        return t * cos_ref[...] + tl * sa_ref[...] + tr * sb_ref[...]

    def compute_chunk(c):
        xc = xg[c]
        q = jnp.dot(xc, wq_ref[...],
                    preferred_element_type=jnp.float32).astype(jnp.bfloat16)
        qb[...] = rope(q)
        k = jnp.dot(xc, wk_ref[...],
                    preferred_element_type=jnp.float32).astype(jnp.bfloat16)
        kb[...] = rope(k)
        vb[...] = jnp.dot(
            xc, wv_ref[...],
            preferred_element_type=jnp.float32).astype(jnp.bfloat16)

        def attn_b(b, _):
            rs = pl.ds(b * SQ, SQ)
            for h in range(HL):
                cs = pl.ds(h * DH, DH)
                qh = qb[rs, cs]
                kh = kb[rs, cs]
                s = lax.dot_general(
                    qh, kh, (((1,), (1,)), ((), ())),
                    preferred_element_type=jnp.float32,
                )
                w = jnp.exp(s)
                r = 1.0 / jnp.sum(w, axis=1, keepdims=True)
                ctx = jnp.dot(w.astype(jnp.bfloat16), vb[rs, cs],
                              preferred_element_type=jnp.float32)
                ctxb[rs, cs] = (ctx * r).astype(jnp.bfloat16)
            return 0

        lax.fori_loop(0, B, attn_b, 0)
        pout[c] = jnp.dot(
            ctxb[...], wo_ref[...], preferred_element_type=jnp.float32
        ).astype(jnp.bfloat16)

    ag_a = ag_hop(0, 3, right)
    ag_b = ag_hop(1, 3, left)
    ag_a.start()
    ag_b.start()
    compute_chunk(3)

    ag_a.wait_recv()
    ag_c = ag_hop(2, 0, right)
    ag_c.start()
    ag_b.wait_recv()
    compute_chunk(1)
    rs0 = rs_hop(0, 1)
    rs0.start()

    ag_c.wait_recv()
    compute_chunk(2)
    rs0.wait_recv()
    pout[2] = pout[2] + rsbuf[0]
    rs1 = rs_hop(1, 2)
    rs1.start()

    compute_chunk(0)
    rs1.wait_recv()
    pout[0] = pout[0] + rsbuf[1]
    rs2 = rs_hop(2, 0)
    rs2.start()

    rs2.wait_recv()
    out_ref[...] = (pout[3] + rsbuf[2]).reshape(B, SQ, D)

    for d in (ag_a, ag_b, ag_c, rs0, rs1, rs2):
        d.wait_send()

    @functools.partial(pl.run_scoped, sem=pltpu.SemaphoreType.REGULAR)
    def _(sem):
        for nbr in (left, right):
            pl.semaphore_signal(sem, inc=1, device_id=(nbr,),
                                device_id_type=pl.DeviceIdType.MESH)
        pl.semaphore_wait(sem, 2)


def kernel(x, Wq, Wk, Wv, Wo):
    cosf, sin_a, sin_b = _rope_tables()
    bf = jnp.bfloat16
    args = (
        x.astype(bf).reshape(R, D),
        (Wq * SCALE).astype(bf),
        Wk.astype(bf),
        Wv.astype(bf),
        Wo.astype(bf),
        jnp.asarray(cosf, dtype=bf),
        jnp.asarray(sin_a, dtype=bf),
        jnp.asarray(sin_b, dtype=bf),
    )
    out = pl.pallas_call(
        _body,
        out_shape=jax.ShapeDtypeStruct((B, SQ, D), bf),
        in_specs=[pl.BlockSpec(memory_space=pltpu.VMEM)] * len(args),
        out_specs=pl.BlockSpec(memory_space=pltpu.VMEM),
        scratch_shapes=[
            pltpu.VMEM((N_DEV, R, D), jnp.bfloat16),
            pltpu.VMEM((N_DEV, R, D), jnp.bfloat16),
            pltpu.VMEM((N_DEV - 1, R, D), jnp.bfloat16),
            pltpu.VMEM((R, D), jnp.bfloat16),
            pltpu.VMEM((R, D), jnp.bfloat16),
            pltpu.VMEM((R, D), jnp.bfloat16),
            pltpu.VMEM((R, D), jnp.bfloat16),
            pltpu.SemaphoreType.DMA((N_DEV - 1,)),
            pltpu.SemaphoreType.DMA((N_DEV - 1,)),
            pltpu.SemaphoreType.DMA((N_DEV - 1,)),
            pltpu.SemaphoreType.DMA((N_DEV - 1,)),
        ],
        compiler_params=pltpu.CompilerParams(
            collective_id=0,
            vmem_limit_bytes=100 * 1024 * 1024,
        ),
    )(*args)
    return out


if __name__ == "__main__":
    t = np.random.randn(R, D).astype(np.float64)
    cos, sa, sb = _rope_tables()
    tl = np.concatenate([t[:, 1:], t[:, :1]], axis=1)
    tr = np.concatenate([t[:, -1:], t[:, :-1]], axis=1)
    mine = t * cos + tl * sa + tr * sb

    t4 = t.reshape(B, SQ, HL, DH)
    inv = 1.0 / (10000.0 ** (np.arange(0, DH, 2) / DH))
    pos = np.arange(SQ)[:, None] * inv[None, :]
    cos_r = np.repeat(np.cos(pos), 2, axis=-1)
    sin_r = np.repeat(np.sin(pos), 2, axis=-1)
    t2 = t4.reshape(B, SQ, HL, DH // 2, 2)
    t_r = np.stack([-t2[..., 1], t2[..., 0]], axis=-1).reshape(B, SQ, HL, DH)
    ref = t4 * cos_r[None, :, None, :] + t_r * sin_r[None, :, None, :]
    print("rope table max err:", np.abs(mine - ref.reshape(R, D)).max())
